# Initial kernel scaffold; baseline (speedup 1.0000x reference)
import jax
import jax.numpy as jnp
from jax.experimental import pallas as pl


def kernel(x, edge_index, W1, b1, W2, b2, W3, b3):
    raise NotImplementedError("write your pallas kernel here")


# speedup vs baseline: 353.3932x; 353.3932x over previous
import functools

import jax
import jax.numpy as jnp
from jax import lax
from jax.experimental import pallas as pl
from jax.experimental.pallas import tpu as pltpu
from jax.experimental.pallas import tpu_sc as plsc

N = 10000
IN_DIM = 128
LAT = 256
E = 320000

NC = 2
NS = 16
NW = NC * NS
LANES = 16

EPW = E // NW
CHUNK_A = 80
NCHUNK_A = EPW // CHUNK_A

ROWS_PER = (N + NW - 1) // NW
CAP = 128
CE = 8000
NCHUNK_B = E // CE
NGRP = CE // LANES


def _lrelu(a):
    return jnp.where(a >= 0, a, 0.01 * a)


def _sload(ref, idx):
    return ref[pl.ds(idx, LANES)][0]


def _encode_body(x_ref, w1_ref, b1_ref, w2_ref, h_ref, g_ref):
    h = _lrelu(jnp.dot(x_ref[...], w1_ref[...],
                       preferred_element_type=jnp.float32) + b1_ref[...])
    h_ref[...] = h
    g_ref[...] = jnp.dot(h, w2_ref[...], preferred_element_type=jnp.float32)


def _encode(x, W1, b1, W2):
    RB = 1000
    grid = (N // RB,)
    return pl.pallas_call(
        _encode_body,
        grid=grid,
        in_specs=[
            pl.BlockSpec((RB, IN_DIM), lambda i: (i, 0)),
            pl.BlockSpec((IN_DIM, LAT), lambda i: (0, 0)),
            pl.BlockSpec((1, LAT), lambda i: (0, 0)),
            pl.BlockSpec((LAT, LAT), lambda i: (0, 0)),
        ],
        out_specs=[
            pl.BlockSpec((RB, LAT), lambda i: (i, 0)),
            pl.BlockSpec((RB, LAT), lambda i: (i, 0)),
        ],
        out_shape=[
            jax.ShapeDtypeStruct((N, LAT), jnp.float32),
            jax.ShapeDtypeStruct((N, LAT), jnp.float32),
        ],
    )(x, W1, b1.reshape(1, LAT), W2)


def _scores_body(g_hbm, src_hbm, dst_hbm, b2_hbm, scores_hbm,
                 idx_u, idx_v, urows, vrows, b2_v, sc_chunk, sem):
    c = lax.axis_index("c")
    s = lax.axis_index("s")
    wid = s * NC + c
    base = wid * EPW
    pltpu.sync_copy(b2_hbm, b2_v)
    iota = lax.iota(jnp.int32, LANES)

    def chunk_body(ci, _):
        eb = base + ci * CHUNK_A
        pltpu.sync_copy(src_hbm.at[pl.ds(eb, CHUNK_A)], idx_u)
        pltpu.sync_copy(dst_hbm.at[pl.ds(eb, CHUNK_A)], idx_v)
        pltpu.async_copy(g_hbm.at[idx_u], urows, sem).wait()
        pltpu.async_copy(g_hbm.at[idx_v], vrows, sem).wait()
        for gg in range(CHUNK_A // LANES):
            def lane_body(l, sumv):
                e = gg * LANES + l
                acc = jnp.zeros((LANES,), jnp.float32)
                for j in range(LAT // LANES):
                    u16 = urows[e, pl.ds(j * LANES, LANES)]
                    v16 = vrows[e, pl.ds(j * LANES, LANES)]
                    b16 = b2_v[pl.ds(j * LANES, LANES)]
                    t = u16 - v16 + b16
                    acc = acc + jnp.where(t >= 0, t, 0.01 * t)
                sedge = jnp.sum(acc)
                return sumv + jnp.where(iota == l, sedge, 0.0)
            sumv = lax.fori_loop(0, LANES, lane_body,
                                 jnp.zeros((LANES,), jnp.float32))
            sig = 1.0 / (1.0 + jnp.exp(-sumv))
            sc_chunk[pl.ds(gg * LANES, LANES)] = sig
        pltpu.sync_copy(sc_chunk, scores_hbm.at[pl.ds(eb, CHUNK_A)])
        return 0

    lax.fori_loop(0, NCHUNK_A, chunk_body, 0)


def _scores(g, e_src, e_dst, b2):
    mesh = plsc.VectorSubcoreMesh(core_axis_name="c", subcore_axis_name="s",
                                  num_cores=NC, num_subcores=NS)
    f = pl.kernel(
        _scores_body,
        out_type=jax.ShapeDtypeStruct((E,), jnp.float32),
        mesh=mesh,
        compiler_params=pltpu.CompilerParams(needs_layout_passes=False),
        scratch_types=[
            pltpu.VMEM((CHUNK_A,), jnp.int32),
            pltpu.VMEM((CHUNK_A,), jnp.int32),
            pltpu.VMEM((CHUNK_A, LAT), jnp.float32),
            pltpu.VMEM((CHUNK_A, LAT), jnp.float32),
            pltpu.VMEM((LAT,), jnp.float32),
            pltpu.VMEM((CHUNK_A,), jnp.float32),
            pltpu.SemaphoreType.DMA,
        ],
    )
    return f(g, e_src, e_dst, b2)


def _rank_body(src_hbm, dst_hbm, scores_hbm, params_hbm, out_hbm,
               src_c, dst_c, scr_c, gdst, gval, cnt, svm, rowbuf, params_v):
    c = lax.axis_index("c")
    s = lax.axis_index("s")
    wid = s * NC + c
    lo = wid * ROWS_PER
    nrows = jnp.minimum(ROWS_PER, N - lo)
    iota = lax.iota(jnp.int32, LANES)
    zeros16 = jnp.zeros((LANES,), jnp.float32)

    pltpu.sync_copy(params_hbm, params_v)
    pv = params_v[...]
    w3s = pv[0]
    b3s = pv[1]

    occ_cal, _ = plsc.scan_count(jnp.zeros((LANES,), jnp.int32))
    occ_base = jnp.min(occ_cal)

    for i in range(320 // LANES):
        cnt[pl.ds(i * LANES, LANES)] = jnp.zeros((LANES,), jnp.int32)
    def zrow(i, _):
        rowbuf[pl.ds(i * LANES, LANES)] = zeros16
        return 0
    lax.fori_loop(0, N // LANES, zrow, 0)

    def chunk_body(ci, _):
        eb = ci * CE
        pltpu.sync_copy(src_hbm.at[pl.ds(eb, CE)], src_c)
        pltpu.sync_copy(dst_hbm.at[pl.ds(eb, CE)], dst_c)
        pltpu.sync_copy(scores_hbm.at[pl.ds(eb, CE)], scr_c)

        def grp_body(gg, _):
            off = gg * LANES
            s16 = src_c[pl.ds(off, LANES)]
            d16 = dst_c[pl.ds(off, LANES)]
            v16 = scr_c[pl.ds(off, LANES)]
            inr = (s16 >= lo) & (s16 < lo + nrows)
            locv = jnp.where(inr, s16 - lo, 0)
            occ, lastm = plsc.scan_count(locv, mask=inr)
            occ0 = occ - occ_base
            old = plsc.load_gather(cnt, [locv], mask=inr)
            pos = old + occ0
            ok = inr & (pos < CAP)
            gidx = jnp.where(ok, locv * CAP + pos, 0)
            plsc.store_scatter(gdst, [gidx], d16, mask=ok)
            plsc.store_scatter(gval, [gidx], v16, mask=ok)
            plsc.addupdate_scatter(cnt, [locv], occ0 + 1, mask=lastm & inr)
            return 0

        lax.fori_loop(0, NGRP, grp_body, 0)
        return 0

    lax.fori_loop(0, NCHUNK_B, chunk_body, 0)

    def row_body(r, _):
        m = jnp.minimum(_sload(cnt, r), CAP)
        o = r * CAP

        acc = zeros16
        for jb in range(CAP // LANES):
            jidx = jb * LANES + iota
            vv = jidx < m
            acc = acc + jnp.where(vv, gval[pl.ds(o + jb * LANES, LANES)], 0.0)
        rs = jnp.sum(acc)
        kk = w3s * rs + b3s
        kk = jnp.where(kk >= 0, kk, 0.01 * kk)

        for jb in range(CAP // LANES):
            @pl.when(m > jb * LANES)
            def _():
                jidx = jb * LANES + iota
                jv = jidx < m
                dstj = gdst[pl.ds(o + jb * LANES, LANES)]

                def b1(j2, carry):
                    sv16, first16 = carry
                    dp = _sload(gdst, o + j2)
                    vp = _sload(gval, o + j2)
                    eq = jv & (dstj == dp)
                    sv16 = sv16 + jnp.where(eq, vp, 0.0)
                    first16 = jnp.minimum(first16,
                                          jnp.where(eq, j2, jnp.int32(1 << 30)))
                    return sv16, first16

                sv16, first16 = lax.fori_loop(
                    0, m, b1,
                    (zeros16, jnp.full((LANES,), 1 << 30, jnp.int32)))
                keep16 = jv & (first16 == jidx)
                svm[pl.ds(jb * LANES, LANES)] = jnp.where(keep16, sv16, -1.0)

        for jb in range(CAP // LANES):
            @pl.when(m > jb * LANES)
            def _():
                dstj = gdst[pl.ds(o + jb * LANES, LANES)]
                svj = svm[pl.ds(jb * LANES, LANES)]
                alive = svj > 0

                def b2(j2, rank16):
                    sp = _sload(svm, j2)
                    dp = _sload(gdst, o + j2)
                    gt = sp > svj
                    tie = (sp == svj) & (dp < dstj)
                    return rank16 + jnp.where(gt | tie, 1.0, 0.0)

                rank16 = lax.fori_loop(0, m, b2, zeros16)
                t16 = rank16 - kk
                e2 = jnp.exp(2.0 * t16)
                th = 1.0 - 2.0 / (e2 + 1.0)
                out16 = svj * (1.5 - 0.5 * th)
                plsc.store_scatter(rowbuf, [dstj], out16, mask=alive)

        pltpu.sync_copy(rowbuf, out_hbm.at[lo + r])

        for jb in range(CAP // LANES):
            @pl.when(m > jb * LANES)
            def _():
                jidx = jb * LANES + iota
                jv = jidx < m
                dstj = gdst[pl.ds(o + jb * LANES, LANES)]
                plsc.store_scatter(rowbuf, [dstj], zeros16, mask=jv)
        return 0

    lax.fori_loop(0, nrows, row_body, 0)


def _rank_scatter(e_src, e_dst, scores, params):
    mesh = plsc.VectorSubcoreMesh(core_axis_name="c", subcore_axis_name="s",
                                  num_cores=NC, num_subcores=NS)
    f = pl.kernel(
        _rank_body,
        out_type=jax.ShapeDtypeStruct((N, N), jnp.float32),
        mesh=mesh,
        compiler_params=pltpu.CompilerParams(needs_layout_passes=False),
        scratch_types=[
            pltpu.VMEM((CE,), jnp.int32),
            pltpu.VMEM((CE,), jnp.int32),
            pltpu.VMEM((CE,), jnp.float32),
            pltpu.VMEM((ROWS_PER * CAP + LANES,), jnp.int32),
            pltpu.VMEM((ROWS_PER * CAP + LANES,), jnp.float32),
            pltpu.VMEM((336,), jnp.int32),
            pltpu.VMEM((CAP + LANES,), jnp.float32),
            pltpu.VMEM((N,), jnp.float32),
            pltpu.VMEM((LANES,), jnp.float32),
        ],
    )
    return f(e_src, e_dst, scores, params)


def kernel(x, edge_index, W1, b1, W2, b2, W3, b3):
    h, g = _encode(x, W1, b1, W2)
    e_src = edge_index[0]
    e_dst = edge_index[1]
    scores = _scores(g, e_src, e_dst, b2)
    params = jnp.zeros((LANES,), jnp.float32)
    params = params.at[0].set(W3[0, 0]).at[1].set(b3[0])
    out = _rank_scatter(e_src, e_dst, scores, params)
    return out, h


# speedup vs baseline: 403.5947x; 1.1421x over previous
import functools

import jax
import jax.numpy as jnp
from jax import lax
from jax.experimental import pallas as pl
from jax.experimental.pallas import tpu as pltpu
from jax.experimental.pallas import tpu_sc as plsc

N = 10000
IN_DIM = 128
LAT = 256
E = 320000

NC = 2
NS = 16
NW = NC * NS
LANES = 16

EPW = E // NW
CHUNK_A = 80
NCHUNK_A = EPW // CHUNK_A

ROWS_PER = (N + NW - 1) // NW
CAP = 128
CE = 8000
NCHUNK_B = E // CE
NGRP = CE // LANES


def _lrelu(a):
    return jnp.where(a >= 0, a, 0.01 * a)


def _sload(ref, idx):
    return ref[pl.ds(idx, LANES)][0]


def _encode_body(x_ref, w1_ref, b1_ref, w2_ref, h_ref, g_ref):
    h = _lrelu(jnp.dot(x_ref[...], w1_ref[...],
                       preferred_element_type=jnp.float32) + b1_ref[...])
    h_ref[...] = h
    g_ref[...] = jnp.dot(h, w2_ref[...], preferred_element_type=jnp.float32)


def _encode(x, W1, b1, W2):
    RB = 1000
    grid = (N // RB,)
    return pl.pallas_call(
        _encode_body,
        grid=grid,
        in_specs=[
            pl.BlockSpec((RB, IN_DIM), lambda i: (i, 0)),
            pl.BlockSpec((IN_DIM, LAT), lambda i: (0, 0)),
            pl.BlockSpec((1, LAT), lambda i: (0, 0)),
            pl.BlockSpec((LAT, LAT), lambda i: (0, 0)),
        ],
        out_specs=[
            pl.BlockSpec((RB, LAT), lambda i: (i, 0)),
            pl.BlockSpec((RB, LAT), lambda i: (i, 0)),
        ],
        out_shape=[
            jax.ShapeDtypeStruct((N, LAT), jnp.float32),
            jax.ShapeDtypeStruct((N, LAT), jnp.float32),
        ],
    )(x, W1, b1.reshape(1, LAT), W2)


def _scores_body(g_hbm, src_hbm, dst_hbm, b2_hbm, scores_hbm,
                 idx_u0, idx_v0, urows0, vrows0,
                 idx_u1, idx_v1, urows1, vrows1,
                 b2_v, sc_chunk, sem0, sem1):
    c = lax.axis_index("c")
    s = lax.axis_index("s")
    wid = s * NC + c
    base = wid * EPW
    pltpu.sync_copy(b2_hbm, b2_v)
    iota = lax.iota(jnp.int32, LANES)
    b2r = [b2_v[pl.ds(j * LANES, LANES)] for j in range(LAT // LANES)]

    bufs = ((idx_u0, idx_v0, urows0, vrows0, sem0),
            (idx_u1, idx_v1, urows1, vrows1, sem1))

    def issue(ci, b):
        iu, iv, ur, vr, sm = bufs[b]
        eb = base + ci * CHUNK_A
        pltpu.sync_copy(src_hbm.at[pl.ds(eb, CHUNK_A)], iu)
        pltpu.sync_copy(dst_hbm.at[pl.ds(eb, CHUNK_A)], iv)
        pltpu.async_copy(g_hbm.at[iu], ur, sm)
        pltpu.async_copy(g_hbm.at[iv], vr, sm)

    def wait(b):
        iu, iv, ur, vr, sm = bufs[b]
        pltpu.make_async_copy(g_hbm.at[iu], ur, sm).wait()
        pltpu.make_async_copy(g_hbm.at[iv], vr, sm).wait()

    def compute(ci, b):
        _, _, ur, vr, _ = bufs[b]
        eb = base + ci * CHUNK_A
        for gg in range(CHUNK_A // LANES):
            def pair_body(l, sumv):
                e0 = gg * LANES + 2 * l
                e1 = e0 + 1
                acc0 = jnp.zeros((LANES,), jnp.float32)
                acc1 = jnp.zeros((LANES,), jnp.float32)
                for j in range(LAT // LANES):
                    sl = pl.ds(j * LANES, LANES)
                    t0 = ur[e0, sl] - vr[e0, sl] + b2r[j]
                    t1 = ur[e1, sl] - vr[e1, sl] + b2r[j]
                    acc0 = acc0 + jnp.where(t0 >= 0, t0, 0.01 * t0)
                    acc1 = acc1 + jnp.where(t1 >= 0, t1, 0.01 * t1)
                s0 = jnp.sum(acc0)
                s1 = jnp.sum(acc1)
                return (sumv + jnp.where(iota == 2 * l, s0, 0.0)
                             + jnp.where(iota == 2 * l + 1, s1, 0.0))
            sumv = lax.fori_loop(0, LANES // 2, pair_body,
                                 jnp.zeros((LANES,), jnp.float32))
            sig = 1.0 / (1.0 + jnp.exp(-sumv))
            sc_chunk[pl.ds(gg * LANES, LANES)] = sig
        pltpu.sync_copy(sc_chunk, scores_hbm.at[pl.ds(eb, CHUNK_A)])

    issue(0, 0)

    def pair_chunks(p, _):
        c0 = 2 * p
        c1 = 2 * p + 1
        wait(0)

        @pl.when(c1 < NCHUNK_A)
        def _():
            issue(c1, 1)
        compute(c0, 0)

        @pl.when(c1 < NCHUNK_A)
        def _():
            wait(1)

            @pl.when(c1 + 1 < NCHUNK_A)
            def _():
                issue(c1 + 1, 0)
            compute(c1, 1)
        return 0

    lax.fori_loop(0, (NCHUNK_A + 1) // 2, pair_chunks, 0)


def _scores(g, e_src, e_dst, b2):
    mesh = plsc.VectorSubcoreMesh(core_axis_name="c", subcore_axis_name="s",
                                  num_cores=NC, num_subcores=NS)
    f = pl.kernel(
        _scores_body,
        out_type=jax.ShapeDtypeStruct((E,), jnp.float32),
        mesh=mesh,
        compiler_params=pltpu.CompilerParams(needs_layout_passes=False),
        scratch_types=[
            pltpu.VMEM((CHUNK_A,), jnp.int32),
            pltpu.VMEM((CHUNK_A,), jnp.int32),
            pltpu.VMEM((CHUNK_A, LAT), jnp.float32),
            pltpu.VMEM((CHUNK_A, LAT), jnp.float32),
            pltpu.VMEM((CHUNK_A,), jnp.int32),
            pltpu.VMEM((CHUNK_A,), jnp.int32),
            pltpu.VMEM((CHUNK_A, LAT), jnp.float32),
            pltpu.VMEM((CHUNK_A, LAT), jnp.float32),
            pltpu.VMEM((LAT,), jnp.float32),
            pltpu.VMEM((CHUNK_A,), jnp.float32),
            pltpu.SemaphoreType.DMA,
            pltpu.SemaphoreType.DMA,
        ],
    )
    return f(g, e_src, e_dst, b2)


def _rank_body(src_hbm, dst_hbm, scores_hbm, params_hbm, out_hbm,
               src_c, dst_c, scr_c, gdst, gval, cnt, svm, rowbuf, params_v):
    c = lax.axis_index("c")
    s = lax.axis_index("s")
    wid = s * NC + c
    lo = wid * ROWS_PER
    nrows = jnp.minimum(ROWS_PER, N - lo)
    iota = lax.iota(jnp.int32, LANES)
    zeros16 = jnp.zeros((LANES,), jnp.float32)

    pltpu.sync_copy(params_hbm, params_v)
    pv = params_v[...]
    w3s = pv[0]
    b3s = pv[1]

    occ_cal, _ = plsc.scan_count(jnp.zeros((LANES,), jnp.int32))
    occ_base = jnp.min(occ_cal)

    for i in range(320 // LANES):
        cnt[pl.ds(i * LANES, LANES)] = jnp.zeros((LANES,), jnp.int32)
    def zrow(i, _):
        rowbuf[pl.ds(i * LANES, LANES)] = zeros16
        return 0
    lax.fori_loop(0, N // LANES, zrow, 0)

    def chunk_body(ci, _):
        eb = ci * CE
        pltpu.sync_copy(src_hbm.at[pl.ds(eb, CE)], src_c)
        pltpu.sync_copy(dst_hbm.at[pl.ds(eb, CE)], dst_c)
        pltpu.sync_copy(scores_hbm.at[pl.ds(eb, CE)], scr_c)

        def grp_body(gg, _):
            off = gg * LANES
            s16 = src_c[pl.ds(off, LANES)]
            d16 = dst_c[pl.ds(off, LANES)]
            v16 = scr_c[pl.ds(off, LANES)]
            inr = (s16 >= lo) & (s16 < lo + nrows)
            locv = jnp.where(inr, s16 - lo, 0)
            occ, lastm = plsc.scan_count(locv, mask=inr)
            occ0 = occ - occ_base
            old = plsc.load_gather(cnt, [locv], mask=inr)
            pos = old + occ0
            ok = inr & (pos < CAP)
            gidx = jnp.where(ok, locv * CAP + pos, 0)
            plsc.store_scatter(gdst, [gidx], d16, mask=ok)
            plsc.store_scatter(gval, [gidx], v16, mask=ok)
            plsc.addupdate_scatter(cnt, [locv], occ0 + 1, mask=lastm & inr)
            return 0

        lax.fori_loop(0, NGRP, grp_body, 0)
        return 0

    lax.fori_loop(0, NCHUNK_B, chunk_body, 0)

    def row_body(r, _):
        m = jnp.minimum(_sload(cnt, r), CAP)
        o = r * CAP

        acc = zeros16
        for jb in range(CAP // LANES):
            jidx = jb * LANES + iota
            vv = jidx < m
            acc = acc + jnp.where(vv, gval[pl.ds(o + jb * LANES, LANES)], 0.0)
        rs = jnp.sum(acc)
        kk = w3s * rs + b3s
        kk = jnp.where(kk >= 0, kk, 0.01 * kk)

        for jb in range(CAP // LANES):
            @pl.when(m > jb * LANES)
            def _():
                jidx = jb * LANES + iota
                jv = jidx < m
                dstj = gdst[pl.ds(o + jb * LANES, LANES)]

                def b1(j2, carry):
                    sv16, first16 = carry
                    dp = _sload(gdst, o + j2)
                    vp = _sload(gval, o + j2)
                    eq = jv & (dstj == dp)
                    sv16 = sv16 + jnp.where(eq, vp, 0.0)
                    first16 = jnp.minimum(first16,
                                          jnp.where(eq, j2, jnp.int32(1 << 30)))
                    return sv16, first16

                sv16, first16 = lax.fori_loop(
                    0, m, b1,
                    (zeros16, jnp.full((LANES,), 1 << 30, jnp.int32)))
                keep16 = jv & (first16 == jidx)
                svm[pl.ds(jb * LANES, LANES)] = jnp.where(keep16, sv16, -1.0)

        for jb in range(CAP // LANES):
            @pl.when(m > jb * LANES)
            def _():
                dstj = gdst[pl.ds(o + jb * LANES, LANES)]
                svj = svm[pl.ds(jb * LANES, LANES)]
                alive = svj > 0

                def b2(j2, rank16):
                    sp = _sload(svm, j2)
                    dp = _sload(gdst, o + j2)
                    gt = sp > svj
                    tie = (sp == svj) & (dp < dstj)
                    return rank16 + jnp.where(gt | tie, 1.0, 0.0)

                rank16 = lax.fori_loop(0, m, b2, zeros16)
                t16 = rank16 - kk
                e2 = jnp.exp(2.0 * t16)
                th = 1.0 - 2.0 / (e2 + 1.0)
                out16 = svj * (1.5 - 0.5 * th)
                plsc.store_scatter(rowbuf, [dstj], out16, mask=alive)

        pltpu.sync_copy(rowbuf, out_hbm.at[lo + r])

        for jb in range(CAP // LANES):
            @pl.when(m > jb * LANES)
            def _():
                jidx = jb * LANES + iota
                jv = jidx < m
                dstj = gdst[pl.ds(o + jb * LANES, LANES)]
                plsc.store_scatter(rowbuf, [dstj], zeros16, mask=jv)
        return 0

    lax.fori_loop(0, nrows, row_body, 0)


def _rank_scatter(e_src, e_dst, scores, params):
    mesh = plsc.VectorSubcoreMesh(core_axis_name="c", subcore_axis_name="s",
                                  num_cores=NC, num_subcores=NS)
    f = pl.kernel(
        _rank_body,
        out_type=jax.ShapeDtypeStruct((N, N), jnp.float32),
        mesh=mesh,
        compiler_params=pltpu.CompilerParams(needs_layout_passes=False),
        scratch_types=[
            pltpu.VMEM((CE,), jnp.int32),
            pltpu.VMEM((CE,), jnp.int32),
            pltpu.VMEM((CE,), jnp.float32),
            pltpu.VMEM((ROWS_PER * CAP + LANES,), jnp.int32),
            pltpu.VMEM((ROWS_PER * CAP + LANES,), jnp.float32),
            pltpu.VMEM((336,), jnp.int32),
            pltpu.VMEM((CAP + LANES,), jnp.float32),
            pltpu.VMEM((N,), jnp.float32),
            pltpu.VMEM((LANES,), jnp.float32),
        ],
    )
    return f(e_src, e_dst, scores, params)


def kernel(x, edge_index, W1, b1, W2, b2, W3, b3):
    h, g = _encode(x, W1, b1, W2)
    e_src = edge_index[0]
    e_dst = edge_index[1]
    scores = _scores(g, e_src, e_dst, b2)
    params = jnp.zeros((LANES,), jnp.float32)
    params = params.at[0].set(W3[0, 0]).at[1].set(b3[0])
    out = _rank_scatter(e_src, e_dst, scores, params)
    return out, h
